# Initial kernel scaffold; baseline (speedup 1.0000x reference)
import jax
import jax.numpy as jnp
from jax.experimental import pallas as pl


def kernel(s0_b, adj_nor, adj_com, w0, b0, gate, wc, bc):
    raise NotImplementedError("write your pallas kernel here")


# speedup vs baseline: 1.5192x; 1.5192x over previous
import jax
import jax.numpy as jnp
from jax.experimental import pallas as pl
from jax.experimental.pallas import tpu as pltpu


def _precompute_kernel(sg_ref, a_ref, c_ref, w0_ref, wc_ref, b0_ref, bc_ref,
                       m_ref, wq_ref, bias_ref):
    a = a_ref[...]
    ca = jnp.dot(c_ref[...], a, preferred_element_type=jnp.float32)
    aca = jnp.dot(a, ca, preferred_element_type=jnp.float32)
    row = jax.lax.broadcasted_iota(jnp.int32, a.shape, 0)
    col = jax.lax.broadcasted_iota(jnp.int32, a.shape, 1)
    eye = jnp.where(row == col, jnp.float32(1.0), jnp.float32(0.0))
    m = sg_ref[0] * eye + sg_ref[1] * a + sg_ref[2] * ca + sg_ref[3] * aca
    m_ref[...] = m
    wq_ref[...] = jnp.dot(w0_ref[...], wc_ref[...],
                          preferred_element_type=jnp.float32)
    bvec = jnp.dot(b0_ref[...], wc_ref[...],
                   preferred_element_type=jnp.float32)
    bias_ref[...] = jnp.sum(m, axis=1, keepdims=True) * bvec + bc_ref[...]


def _batched_kernel(x_ref, m_ref, wq_ref, bias_ref, out_ref):
    t = jnp.dot(x_ref[0], wq_ref[...], preferred_element_type=jnp.float32)
    logits = jnp.dot(m_ref[...], t,
                     preferred_element_type=jnp.float32) + bias_ref[...]
    mx = jnp.max(logits, axis=-1, keepdims=True)
    lse = jnp.log(jnp.sum(jnp.exp(logits - mx), axis=-1, keepdims=True)) + mx
    out_ref[0] = logits - lse


def kernel(s0_b, adj_nor, adj_com, w0, b0, gate, wc, bc):
    B, N, feat = s0_b.shape
    num_classes = wc.shape[1]

    sg = jax.nn.sigmoid(gate.reshape(-1)).astype(jnp.float32)
    b0r = b0.reshape(1, -1)
    bcr = bc.reshape(1, -1)

    m, wq, bias = pl.pallas_call(
        _precompute_kernel,
        out_shape=(
            jax.ShapeDtypeStruct((N, N), jnp.float32),
            jax.ShapeDtypeStruct((feat, num_classes), jnp.float32),
            jax.ShapeDtypeStruct((N, num_classes), jnp.float32),
        ),
        in_specs=[
            pl.BlockSpec(memory_space=pltpu.MemorySpace.SMEM),
            pl.BlockSpec((N, N), lambda: (0, 0)),
            pl.BlockSpec((N, N), lambda: (0, 0)),
            pl.BlockSpec((feat, w0.shape[1]), lambda: (0, 0)),
            pl.BlockSpec((w0.shape[1], num_classes), lambda: (0, 0)),
            pl.BlockSpec((1, w0.shape[1]), lambda: (0, 0)),
            pl.BlockSpec((1, num_classes), lambda: (0, 0)),
        ],
        out_specs=(
            pl.BlockSpec((N, N), lambda: (0, 0)),
            pl.BlockSpec((feat, num_classes), lambda: (0, 0)),
            pl.BlockSpec((N, num_classes), lambda: (0, 0)),
        ),
        cost_estimate=pl.CostEstimate(
            flops=int(2 * 2 * N * N * N),
            transcendentals=0,
            bytes_accessed=int(4 * (3 * N * N + 2 * feat * num_classes))),
    )(sg, adj_nor, adj_com, w0, wc, b0r, bcr)

    flops = int(2 * B * (N * feat * num_classes + N * N * num_classes))
    out = pl.pallas_call(
        _batched_kernel,
        out_shape=jax.ShapeDtypeStruct((B, N, num_classes), jnp.float32),
        grid=(B,),
        in_specs=[
            pl.BlockSpec((1, N, feat), lambda b: (b, 0, 0)),
            pl.BlockSpec((N, N), lambda b: (0, 0)),
            pl.BlockSpec((feat, num_classes), lambda b: (0, 0)),
            pl.BlockSpec((N, num_classes), lambda b: (0, 0)),
        ],
        out_specs=pl.BlockSpec((1, N, num_classes), lambda b: (b, 0, 0)),
        compiler_params=pltpu.CompilerParams(
            dimension_semantics=("parallel",)),
        cost_estimate=pl.CostEstimate(
            flops=flops,
            transcendentals=int(B * N * num_classes + B * N),
            bytes_accessed=int(4 * (s0_b.size + N * N + feat * num_classes
                                    + N * num_classes + B * N * num_classes))),
    )(s0_b, m, wq, bias)

    return out


# speedup vs baseline: 3.1005x; 2.0409x over previous
import jax
import jax.numpy as jnp
from jax.experimental import pallas as pl
from jax.experimental.pallas import tpu as pltpu


def _precompute_kernel(sg_ref, a_ref, c_ref, w0_ref, wc_ref, b0_ref, bc_ref,
                       m_ref, wq_ref, bias_ref):
    a = a_ref[...]
    ca = jnp.dot(c_ref[...], a, preferred_element_type=jnp.float32)
    aca = jnp.dot(a, ca, preferred_element_type=jnp.float32)
    row = jax.lax.broadcasted_iota(jnp.int32, a.shape, 0)
    col = jax.lax.broadcasted_iota(jnp.int32, a.shape, 1)
    eye = jnp.where(row == col, jnp.float32(1.0), jnp.float32(0.0))
    m = sg_ref[0] * eye + sg_ref[1] * a + sg_ref[2] * ca + sg_ref[3] * aca
    m_ref[...] = m.astype(jnp.bfloat16)
    wq_ref[...] = jnp.dot(w0_ref[...], wc_ref[...],
                          preferred_element_type=jnp.float32
                          ).astype(jnp.bfloat16)
    bvec = jnp.dot(b0_ref[...], wc_ref[...],
                   preferred_element_type=jnp.float32)
    bias_ref[...] = jnp.sum(m, axis=1, keepdims=True) * bvec + bc_ref[...]


def _batched_kernel(x_ref, m_ref, wq_ref, bias_ref, out_ref):
    g, n, feat = x_ref.shape
    c = wq_ref.shape[1]
    xb = x_ref[...].reshape(g * n, feat).astype(jnp.bfloat16)
    t = jnp.dot(xb, wq_ref[...], preferred_element_type=jnp.float32)
    tw = jnp.concatenate([t[i * n:(i + 1) * n] for i in range(g)],
                         axis=1).astype(jnp.bfloat16)
    y = jnp.dot(m_ref[...], tw, preferred_element_type=jnp.float32)
    bias = bias_ref[...]
    for i in range(g):
        logits = y[:, i * c:(i + 1) * c] + bias
        mx = jnp.max(logits, axis=-1, keepdims=True)
        lse = jnp.log(jnp.sum(jnp.exp(logits - mx), axis=-1,
                              keepdims=True)) + mx
        out_ref[i] = logits - lse


def kernel(s0_b, adj_nor, adj_com, w0, b0, gate, wc, bc):
    B, N, feat = s0_b.shape
    num_classes = wc.shape[1]

    sg = jax.nn.sigmoid(gate.reshape(-1)).astype(jnp.float32)
    b0r = b0.reshape(1, -1)
    bcr = bc.reshape(1, -1)

    m, wq, bias = pl.pallas_call(
        _precompute_kernel,
        out_shape=(
            jax.ShapeDtypeStruct((N, N), jnp.bfloat16),
            jax.ShapeDtypeStruct((feat, num_classes), jnp.bfloat16),
            jax.ShapeDtypeStruct((N, num_classes), jnp.float32),
        ),
        in_specs=[
            pl.BlockSpec(memory_space=pltpu.MemorySpace.SMEM),
            pl.BlockSpec((N, N), lambda: (0, 0)),
            pl.BlockSpec((N, N), lambda: (0, 0)),
            pl.BlockSpec((feat, w0.shape[1]), lambda: (0, 0)),
            pl.BlockSpec((w0.shape[1], num_classes), lambda: (0, 0)),
            pl.BlockSpec((1, w0.shape[1]), lambda: (0, 0)),
            pl.BlockSpec((1, num_classes), lambda: (0, 0)),
        ],
        out_specs=(
            pl.BlockSpec((N, N), lambda: (0, 0)),
            pl.BlockSpec((feat, num_classes), lambda: (0, 0)),
            pl.BlockSpec((N, num_classes), lambda: (0, 0)),
        ),
        cost_estimate=pl.CostEstimate(
            flops=int(2 * 2 * N * N * N),
            transcendentals=0,
            bytes_accessed=int(4 * (3 * N * N + 2 * feat * num_classes))),
    )(sg, adj_nor, adj_com, w0, wc, b0r, bcr)

    G = 4 if B % 4 == 0 else 1
    flops = int(2 * B * (N * feat * num_classes + N * N * num_classes))
    out = pl.pallas_call(
        _batched_kernel,
        out_shape=jax.ShapeDtypeStruct((B, N, num_classes), jnp.float32),
        grid=(B // G,),
        in_specs=[
            pl.BlockSpec((G, N, feat), lambda b: (b, 0, 0)),
            pl.BlockSpec((N, N), lambda b: (0, 0)),
            pl.BlockSpec((feat, num_classes), lambda b: (0, 0)),
            pl.BlockSpec((N, num_classes), lambda b: (0, 0)),
        ],
        out_specs=pl.BlockSpec((G, N, num_classes), lambda b: (b, 0, 0)),
        compiler_params=pltpu.CompilerParams(
            dimension_semantics=("parallel",)),
        cost_estimate=pl.CostEstimate(
            flops=flops,
            transcendentals=int(B * N * num_classes + B * N),
            bytes_accessed=int(4 * (s0_b.size + N * N + feat * num_classes
                                    + N * num_classes + B * N * num_classes))),
    )(s0_b, m, wq, bias)

    return out


# speedup vs baseline: 3.8569x; 1.2440x over previous
import jax
import jax.numpy as jnp
from jax.experimental import pallas as pl
from jax.experimental.pallas import tpu as pltpu


def _precompute_kernel(sg_ref, a_ref, c_ref, w0_ref, wc_ref, b0_ref, bc_ref,
                       m_ref, wq_ref, bias_ref):
    a = a_ref[...]
    ca = jnp.dot(c_ref[...], a, preferred_element_type=jnp.float32)
    aca = jnp.dot(a, ca, preferred_element_type=jnp.float32)
    row = jax.lax.broadcasted_iota(jnp.int32, a.shape, 0)
    col = jax.lax.broadcasted_iota(jnp.int32, a.shape, 1)
    eye = jnp.where(row == col, jnp.float32(1.0), jnp.float32(0.0))
    m = sg_ref[0] * eye + sg_ref[1] * a + sg_ref[2] * ca + sg_ref[3] * aca
    m_ref[...] = m.astype(jnp.bfloat16)
    wq_ref[...] = jnp.dot(w0_ref[...], wc_ref[...],
                          preferred_element_type=jnp.float32
                          ).astype(jnp.bfloat16)
    bvec = jnp.dot(b0_ref[...], wc_ref[...],
                   preferred_element_type=jnp.float32)
    bias_ref[...] = jnp.sum(m, axis=1, keepdims=True) * bvec + bc_ref[...]


def _batched_kernel(x_ref, m_ref, wq_ref, bias_ref, out_ref):
    g, n, feat = x_ref.shape
    c = wq_ref.shape[1]
    xb = x_ref[...].reshape(g * n, feat).astype(jnp.bfloat16)
    t = jnp.dot(xb, wq_ref[...], preferred_element_type=jnp.float32)
    tw = jnp.concatenate([t[i * n:(i + 1) * n] for i in range(g)],
                         axis=1).astype(jnp.bfloat16)
    y = jnp.dot(m_ref[...], tw, preferred_element_type=jnp.float32)
    bias = bias_ref[...]
    for i in range(g):
        logits = y[:, i * c:(i + 1) * c] + bias
        mx = jnp.max(logits, axis=-1, keepdims=True)
        lse = jnp.log(jnp.sum(jnp.exp(logits - mx), axis=-1,
                              keepdims=True)) + mx
        out_ref[i] = logits - lse


def kernel(s0_b, adj_nor, adj_com, w0, b0, gate, wc, bc):
    B, N, feat = s0_b.shape
    num_classes = wc.shape[1]

    sg = jax.nn.sigmoid(gate.reshape(-1)).astype(jnp.float32)
    b0r = b0.reshape(1, -1)
    bcr = bc.reshape(1, -1)

    m, wq, bias = pl.pallas_call(
        _precompute_kernel,
        out_shape=(
            jax.ShapeDtypeStruct((N, N), jnp.bfloat16),
            jax.ShapeDtypeStruct((feat, num_classes), jnp.bfloat16),
            jax.ShapeDtypeStruct((N, num_classes), jnp.float32),
        ),
        in_specs=[
            pl.BlockSpec(memory_space=pltpu.MemorySpace.SMEM),
            pl.BlockSpec((N, N), lambda: (0, 0)),
            pl.BlockSpec((N, N), lambda: (0, 0)),
            pl.BlockSpec((feat, w0.shape[1]), lambda: (0, 0)),
            pl.BlockSpec((w0.shape[1], num_classes), lambda: (0, 0)),
            pl.BlockSpec((1, w0.shape[1]), lambda: (0, 0)),
            pl.BlockSpec((1, num_classes), lambda: (0, 0)),
        ],
        out_specs=(
            pl.BlockSpec((N, N), lambda: (0, 0)),
            pl.BlockSpec((feat, num_classes), lambda: (0, 0)),
            pl.BlockSpec((N, num_classes), lambda: (0, 0)),
        ),
        cost_estimate=pl.CostEstimate(
            flops=int(2 * 2 * N * N * N),
            transcendentals=0,
            bytes_accessed=int(4 * (3 * N * N + 2 * feat * num_classes))),
    )(sg, adj_nor, adj_com, w0, wc, b0r, bcr)

    G = 8 if B % 8 == 0 else 1
    flops = int(2 * B * (N * feat * num_classes + N * N * num_classes))
    out = pl.pallas_call(
        _batched_kernel,
        out_shape=jax.ShapeDtypeStruct((B, N, num_classes), jnp.float32),
        grid=(B // G,),
        in_specs=[
            pl.BlockSpec((G, N, feat), lambda b: (b, 0, 0)),
            pl.BlockSpec((N, N), lambda b: (0, 0)),
            pl.BlockSpec((feat, num_classes), lambda b: (0, 0)),
            pl.BlockSpec((N, num_classes), lambda b: (0, 0)),
        ],
        out_specs=pl.BlockSpec((G, N, num_classes), lambda b: (b, 0, 0)),
        compiler_params=pltpu.CompilerParams(
            dimension_semantics=("parallel",)),
        cost_estimate=pl.CostEstimate(
            flops=flops,
            transcendentals=int(B * N * num_classes + B * N),
            bytes_accessed=int(4 * (s0_b.size + N * N + feat * num_classes
                                    + N * num_classes + B * N * num_classes))),
    )(s0_b, m, wq, bias)

    return out


# speedup vs baseline: 4.1288x; 1.0705x over previous
import jax
import jax.numpy as jnp
from jax.experimental import pallas as pl
from jax.experimental.pallas import tpu as pltpu


def _precompute_kernel(sg_ref, a_ref, c_ref, w0_ref, wc_ref, b0_ref, bc_ref,
                       m_ref, wq_ref, bias_ref):
    a = a_ref[...]
    ca = jnp.dot(c_ref[...], a, preferred_element_type=jnp.float32)
    aca = jnp.dot(a, ca, preferred_element_type=jnp.float32)
    row = jax.lax.broadcasted_iota(jnp.int32, a.shape, 0)
    col = jax.lax.broadcasted_iota(jnp.int32, a.shape, 1)
    eye = jnp.where(row == col, jnp.float32(1.0), jnp.float32(0.0))
    m = sg_ref[0] * eye + sg_ref[1] * a + sg_ref[2] * ca + sg_ref[3] * aca
    m_ref[...] = m.astype(jnp.bfloat16)
    wq_ref[...] = jnp.dot(w0_ref[...], wc_ref[...],
                          preferred_element_type=jnp.float32
                          ).astype(jnp.bfloat16)
    bvec = jnp.dot(b0_ref[...], wc_ref[...],
                   preferred_element_type=jnp.float32)
    bias_ref[...] = jnp.sum(m, axis=1, keepdims=True) * bvec + bc_ref[...]


def _batched_kernel(x_ref, m_ref, wq_ref, bias_ref, out_ref):
    g, n, feat = x_ref.shape
    c = wq_ref.shape[1]
    xb = x_ref[...].reshape(g * n, feat).astype(jnp.bfloat16)
    t = jnp.dot(xb, wq_ref[...], preferred_element_type=jnp.float32)
    tw = jnp.concatenate([t[i * n:(i + 1) * n] for i in range(g)],
                         axis=1).astype(jnp.bfloat16)
    y = jnp.dot(m_ref[...], tw, preferred_element_type=jnp.float32)
    bias = bias_ref[...]
    for i in range(g):
        logits = y[:, i * c:(i + 1) * c] + bias
        mx = jnp.max(logits, axis=-1, keepdims=True)
        lse = jnp.log(jnp.sum(jnp.exp(logits - mx), axis=-1,
                              keepdims=True)) + mx
        out_ref[i] = logits - lse


def kernel(s0_b, adj_nor, adj_com, w0, b0, gate, wc, bc):
    B, N, feat = s0_b.shape
    num_classes = wc.shape[1]

    sg = jax.nn.sigmoid(gate.reshape(-1)).astype(jnp.float32)
    b0r = b0.reshape(1, -1)
    bcr = bc.reshape(1, -1)

    m, wq, bias = pl.pallas_call(
        _precompute_kernel,
        out_shape=(
            jax.ShapeDtypeStruct((N, N), jnp.bfloat16),
            jax.ShapeDtypeStruct((feat, num_classes), jnp.bfloat16),
            jax.ShapeDtypeStruct((N, num_classes), jnp.float32),
        ),
        in_specs=[
            pl.BlockSpec(memory_space=pltpu.MemorySpace.SMEM),
            pl.BlockSpec((N, N), lambda: (0, 0)),
            pl.BlockSpec((N, N), lambda: (0, 0)),
            pl.BlockSpec((feat, w0.shape[1]), lambda: (0, 0)),
            pl.BlockSpec((w0.shape[1], num_classes), lambda: (0, 0)),
            pl.BlockSpec((1, w0.shape[1]), lambda: (0, 0)),
            pl.BlockSpec((1, num_classes), lambda: (0, 0)),
        ],
        out_specs=(
            pl.BlockSpec((N, N), lambda: (0, 0)),
            pl.BlockSpec((feat, num_classes), lambda: (0, 0)),
            pl.BlockSpec((N, num_classes), lambda: (0, 0)),
        ),
        cost_estimate=pl.CostEstimate(
            flops=int(2 * 2 * N * N * N),
            transcendentals=0,
            bytes_accessed=int(4 * (3 * N * N + 2 * feat * num_classes))),
    )(sg, adj_nor, adj_com, w0, wc, b0r, bcr)

    G = 16 if B % 16 == 0 else 1
    flops = int(2 * B * (N * feat * num_classes + N * N * num_classes))
    out = pl.pallas_call(
        _batched_kernel,
        out_shape=jax.ShapeDtypeStruct((B, N, num_classes), jnp.float32),
        grid=(B // G,),
        in_specs=[
            pl.BlockSpec((G, N, feat), lambda b: (b, 0, 0)),
            pl.BlockSpec((N, N), lambda b: (0, 0)),
            pl.BlockSpec((feat, num_classes), lambda b: (0, 0)),
            pl.BlockSpec((N, num_classes), lambda b: (0, 0)),
        ],
        out_specs=pl.BlockSpec((G, N, num_classes), lambda b: (b, 0, 0)),
        compiler_params=pltpu.CompilerParams(
            dimension_semantics=("parallel",)),
        cost_estimate=pl.CostEstimate(
            flops=flops,
            transcendentals=int(B * N * num_classes + B * N),
            bytes_accessed=int(4 * (s0_b.size + N * N + feat * num_classes
                                    + N * num_classes + B * N * num_classes))),
    )(s0_b, m, wq, bias)

    return out


# speedup vs baseline: 4.3075x; 1.0433x over previous
import jax
import jax.numpy as jnp
from jax.experimental import pallas as pl
from jax.experimental.pallas import tpu as pltpu


def _fused_kernel(sg_ref, a_ref, c_ref, w0_ref, wc_ref, b0_ref, bc_ref,
                  x_ref, out_ref):
    a = a_ref[...]
    ca = jnp.dot(c_ref[...], a, preferred_element_type=jnp.float32)
    aca = jnp.dot(a, ca, preferred_element_type=jnp.float32)
    row = jax.lax.broadcasted_iota(jnp.int32, a.shape, 0)
    col = jax.lax.broadcasted_iota(jnp.int32, a.shape, 1)
    eye = jnp.where(row == col, jnp.float32(1.0), jnp.float32(0.0))
    m = sg_ref[0] * eye + sg_ref[1] * a + sg_ref[2] * ca + sg_ref[3] * aca
    mb = m.astype(jnp.bfloat16)
    wq = jnp.dot(w0_ref[...], wc_ref[...],
                 preferred_element_type=jnp.float32).astype(jnp.bfloat16)
    bvec = jnp.dot(b0_ref[...], wc_ref[...],
                   preferred_element_type=jnp.float32)
    bias = jnp.sum(m, axis=1, keepdims=True) * bvec + bc_ref[...]

    g, n, feat = x_ref.shape
    c = wc_ref.shape[1]
    xb = x_ref[...].reshape(g * n, feat).astype(jnp.bfloat16)
    t = jnp.dot(xb, wq, preferred_element_type=jnp.float32)
    tw = jnp.concatenate([t[i * n:(i + 1) * n] for i in range(g)],
                         axis=1).astype(jnp.bfloat16)
    y = jnp.dot(mb, tw, preferred_element_type=jnp.float32)
    for i in range(g):
        logits = y[:, i * c:(i + 1) * c] + bias
        mx = jnp.max(logits, axis=-1, keepdims=True)
        lse = jnp.log(jnp.sum(jnp.exp(logits - mx), axis=-1,
                              keepdims=True)) + mx
        out_ref[i] = logits - lse


def kernel(s0_b, adj_nor, adj_com, w0, b0, gate, wc, bc):
    B, N, feat = s0_b.shape
    hid = w0.shape[1]
    num_classes = wc.shape[1]

    sg = jax.nn.sigmoid(gate.reshape(-1)).astype(jnp.float32)
    b0r = b0.reshape(1, -1)
    bcr = bc.reshape(1, -1)

    G = 16 if B % 16 == 0 else 1
    flops = int(2 * B * (N * feat * num_classes + N * N * num_classes)
                + (B // G) * 2 * 2 * N * N * N)
    out = pl.pallas_call(
        _fused_kernel,
        out_shape=jax.ShapeDtypeStruct((B, N, num_classes), jnp.float32),
        grid=(B // G,),
        in_specs=[
            pl.BlockSpec(memory_space=pltpu.MemorySpace.SMEM),
            pl.BlockSpec((N, N), lambda b: (0, 0)),
            pl.BlockSpec((N, N), lambda b: (0, 0)),
            pl.BlockSpec((feat, hid), lambda b: (0, 0)),
            pl.BlockSpec((hid, num_classes), lambda b: (0, 0)),
            pl.BlockSpec((1, hid), lambda b: (0, 0)),
            pl.BlockSpec((1, num_classes), lambda b: (0, 0)),
            pl.BlockSpec((G, N, feat), lambda b: (b, 0, 0)),
        ],
        out_specs=pl.BlockSpec((G, N, num_classes), lambda b: (b, 0, 0)),
        compiler_params=pltpu.CompilerParams(
            dimension_semantics=("parallel",)),
        cost_estimate=pl.CostEstimate(
            flops=flops,
            transcendentals=int(B * N * num_classes + B * N),
            bytes_accessed=int(4 * (s0_b.size + 2 * N * N + w0.size
                                    + wc.size + B * N * num_classes))),
    )(sg, adj_nor, adj_com, w0, wc, b0r, bcr, s0_b)

    return out
